# baseline (speedup 1.0000x reference)
import jax
import jax.numpy as jnp
from jax import lax
from jax.experimental import pallas as pl
from jax.experimental.pallas import tpu as pltpu
from jax.experimental.pallas import tpu_sc as plsc

_N = 10000
_E = 320000
_NF = 128
_NH = 256
_NC = 16
_K = 10

_NS = 16
_NW = 32
_NPAD = 10240
_RPW = _NPAD // _NW
_ZPT = _NPAD // _NS
_CHUNK = 128
_CPW = 80
_EPW = _CPW * _CHUNK
_EREAL = _E // _NW
_NBUF = 8
_GRP = _CPW // _NBUF


def _mlp_body(x_ref, w1_ref, b1_ref, w2_ref, b2_ref, o_ref):
    h = jnp.dot(x_ref[...], w1_ref[...], preferred_element_type=jnp.float32)
    h = jnp.maximum(h + b1_ref[...], 0.0)
    o_ref[...] = jnp.dot(h, w2_ref[...], preferred_element_type=jnp.float32) + b2_ref[...]


_MLP_BLK = 1024


def _mlp(xp, W1, b1, W2, b2):
    return pl.pallas_call(
        _mlp_body,
        grid=(_NPAD // _MLP_BLK,),
        in_specs=[
            pl.BlockSpec((_MLP_BLK, _NF), lambda i: (i, 0)),
            pl.BlockSpec((_NF, _NH), lambda i: (0, 0)),
            pl.BlockSpec((1, _NH), lambda i: (0, 0)),
            pl.BlockSpec((_NH, _NC), lambda i: (0, 0)),
            pl.BlockSpec((1, _NC), lambda i: (0, 0)),
        ],
        out_specs=pl.BlockSpec((_MLP_BLK, _NC), lambda i: (i, 0)),
        out_shape=jax.ShapeDtypeStruct((_NPAD, _NC), jnp.float32),
    )(xp, W1, b1, W2, b2)


def _lsm_body(h_ref, o_ref):
    v = h_ref[...]
    m = jnp.max(v, axis=1, keepdims=True)
    e = jnp.exp(v - m)
    s = jnp.sum(e, axis=1, keepdims=True)
    o_ref[...] = (v - m) - jnp.log(s)


_LSM_BLK = 2000


def _logsoftmax(h):
    return pl.pallas_call(
        _lsm_body,
        grid=(_N // _LSM_BLK,),
        in_specs=[pl.BlockSpec((_LSM_BLK, _NC), lambda i: (i, 0))],
        out_specs=pl.BlockSpec((_LSM_BLK, _NC), lambda i: (i, 0)),
        out_shape=jax.ShapeDtypeStruct((_N, _NC), jnp.float32),
    )(h)


def _sc_body(rows_hbm, cols_hbm, h0_hbm, temp_hbm,
             hidden_hbm, gwork_hbm, part_hbm,
             row_t, col_t, bufs, g_t, acc_t, d2_t, sbuf_t, pbuf_t, zero_t,
             temp_v, s_sh, gsem, ssem, xsem):
    cid = lax.axis_index("c")
    sid = lax.axis_index("s")
    w = cid * _NS + sid
    r0 = w * _RPW
    own = pl.ds(r0, _RPW)
    pr0 = ((1 - cid) * _NS + sid) * _RPW
    peer = pl.ds(pr0, _RPW)
    zrows = pl.ds(sid * _ZPT, _ZPT)

    def xbarrier():
        plsc.subcore_barrier()

        @pl.when(sid == 0)
        def _x():
            pltpu.semaphore_signal(xsem, 1, core_index=1 - cid)
            pl.semaphore_wait(xsem, 1)
        plsc.subcore_barrier()

    pltpu.sync_copy(rows_hbm.at[w], row_t)
    pltpu.sync_copy(cols_hbm.at[w], col_t)
    pltpu.sync_copy(temp_hbm, temp_v)

    ones = jnp.ones((_NC,), jnp.float32)
    zeros = jnp.zeros((_NC,), jnp.float32)

    def _fill_zero(i, c):
        zero_t[i, :] = zeros
        return c

    def _fill_one(i, c):
        zero_t[i, :] = ones
        return c

    def _fill_src_ones(i, c):
        sbuf_t[i, :] = ones
        return c

    lax.fori_loop(0, _CHUNK, _fill_src_ones, 0)

    @pl.when(cid == 0)
    def _i0():
        lax.fori_loop(0, _ZPT, _fill_one, 0)

    @pl.when(cid == 1)
    def _i1():
        lax.fori_loop(0, _ZPT, _fill_zero, 0)
    pltpu.sync_copy(zero_t, s_sh.at[zrows])

    @pl.when(cid == 0)
    def _i0z():
        lax.fori_loop(0, _ZPT, _fill_zero, 0)
    plsc.subcore_barrier()

    def _deg_group(m, c):
        for b in range(_NBUF):
            j = m * _NBUF + b
            pltpu.async_copy(sbuf_t.at[pl.ds(0, _CHUNK)],
                             s_sh.at[col_t.at[j]], ssem.at[0], add=True)
        for b in range(_NBUF):
            j = m * _NBUF + b
            pltpu.make_async_copy(sbuf_t.at[pl.ds(0, _CHUNK)],
                                  s_sh.at[col_t.at[j]], ssem.at[0]).wait()
        return c
    lax.fori_loop(0, _GRP, _deg_group, 0)
    plsc.subcore_barrier()

    pltpu.async_copy(s_sh.at[peer], part_hbm.at[peer], gsem.at[0])
    pltpu.async_copy(s_sh.at[own], sbuf_t, gsem.at[1])
    pltpu.make_async_copy(s_sh.at[peer], part_hbm.at[peer], gsem.at[0]).wait()
    pltpu.make_async_copy(s_sh.at[own], sbuf_t, gsem.at[1]).wait()
    pltpu.async_copy(zero_t.at[pl.ds(0, _RPW)], s_sh.at[own], gsem.at[2])
    pltpu.async_copy(zero_t.at[pl.ds(0, _RPW)], s_sh.at[peer], gsem.at[3])
    pltpu.make_async_copy(zero_t.at[pl.ds(0, _RPW)], s_sh.at[own], gsem.at[2]).wait()
    pltpu.make_async_copy(zero_t.at[pl.ds(0, _RPW)], s_sh.at[peer], gsem.at[3]).wait()
    xbarrier()

    pltpu.sync_copy(part_hbm.at[own], pbuf_t)
    pltpu.sync_copy(h0_hbm.at[own], g_t)
    t0 = temp_v[0, :]

    def _prep(i, c):
        d = sbuf_t[i, :] + pbuf_t[i, :]
        y = 1.0 / d
        for _ in range(20):
            y = y * (1.5 - 0.5 * d * y * y)
        d2_t[i, :] = y * y
        g0 = y * g_t[i, :]
        g_t[i, :] = g0
        acc_t[i, :] = t0 * g0
        return c
    lax.fori_loop(0, _RPW, _prep, 0)
    pltpu.sync_copy(g_t, gwork_hbm.at[own])
    xbarrier()

    def _hop(k, c):
        for b in range(_NBUF):
            pltpu.async_copy(gwork_hbm.at[row_t.at[b]], bufs.at[b], gsem.at[b])

        def _group(m, cc):
            for b in range(_NBUF):
                j = m * _NBUF + b
                pltpu.make_async_copy(gwork_hbm.at[row_t.at[j]],
                                      bufs.at[b], gsem.at[b]).wait()
                pltpu.async_copy(bufs.at[b], s_sh.at[col_t.at[j]],
                                 ssem.at[b], add=True)
            for b in range(_NBUF):
                j = m * _NBUF + b

                @pl.when(m < _GRP - 1)
                def _refill():
                    pltpu.make_async_copy(bufs.at[b], s_sh.at[col_t.at[j]],
                                          ssem.at[b]).wait()
                    pltpu.async_copy(gwork_hbm.at[row_t.at[j + _NBUF]],
                                     bufs.at[b], gsem.at[b])
            return cc
        lax.fori_loop(0, _GRP, _group, 0)
        for b in range(_NBUF):
            j = (_GRP - 1) * _NBUF + b
            pltpu.make_async_copy(bufs.at[b], s_sh.at[col_t.at[j]],
                                  ssem.at[b]).wait()
        plsc.subcore_barrier()

        pltpu.async_copy(s_sh.at[peer], part_hbm.at[peer], gsem.at[0])
        pltpu.async_copy(s_sh.at[own], sbuf_t, gsem.at[1])
        pltpu.make_async_copy(s_sh.at[peer], part_hbm.at[peer],
                              gsem.at[0]).wait()
        pltpu.make_async_copy(s_sh.at[own], sbuf_t, gsem.at[1]).wait()
        pltpu.async_copy(zero_t.at[pl.ds(0, _RPW)], s_sh.at[own], gsem.at[2])
        pltpu.async_copy(zero_t.at[pl.ds(0, _RPW)], s_sh.at[peer], gsem.at[3])
        pltpu.make_async_copy(zero_t.at[pl.ds(0, _RPW)], s_sh.at[own], gsem.at[2]).wait()
        pltpu.make_async_copy(zero_t.at[pl.ds(0, _RPW)], s_sh.at[peer], gsem.at[3]).wait()
        xbarrier()

        pltpu.async_copy(part_hbm.at[own], pbuf_t, gsem.at[0])
        tk = temp_v[k + 1, :]

        def _upd_local(i, cc):
            sbuf_t[i, :] = d2_t[i, :] * (sbuf_t[i, :] + g_t[i, :])
            return cc
        lax.fori_loop(0, _RPW, _upd_local, 0)
        pltpu.make_async_copy(part_hbm.at[own], pbuf_t, gsem.at[0]).wait()

        def _upd_peer(i, cc):
            g_t[i, :] = sbuf_t[i, :] + d2_t[i, :] * pbuf_t[i, :]
            return cc
        lax.fori_loop(0, _RPW, _upd_peer, 0)
        pltpu.async_copy(g_t, gwork_hbm.at[own], gsem.at[1])

        def _upd_acc(i, cc):
            acc_t[i, :] = acc_t[i, :] + tk * g_t[i, :]
            return cc
        lax.fori_loop(0, _RPW, _upd_acc, 0)
        pltpu.make_async_copy(g_t, gwork_hbm.at[own], gsem.at[1]).wait()
        xbarrier()
        return c
    lax.fori_loop(0, _K, _hop, 0)

    def _fin(i, c):
        d2 = d2_t[i, :]
        y = jnp.ones((_NC,), jnp.float32)
        for _ in range(20):
            y = y * (1.5 - 0.5 * d2 * y * y)
        acc_t[i, :] = y * acc_t[i, :]
        return c
    lax.fori_loop(0, _RPW, _fin, 0)
    pltpu.sync_copy(acc_t, hidden_hbm.at[own])


_sc_mesh = plsc.VectorSubcoreMesh(core_axis_name="c", subcore_axis_name="s",
                                  num_cores=2, num_subcores=_NS)

_scprop = pl.kernel(
    _sc_body,
    out_type=(jax.ShapeDtypeStruct((_NPAD, _NC), jnp.float32),
              jax.ShapeDtypeStruct((_NPAD, _NC), jnp.float32),
              jax.ShapeDtypeStruct((_NPAD, _NC), jnp.float32)),
    mesh=_sc_mesh,
    scratch_types=[
        pltpu.VMEM((_CPW, _CHUNK), jnp.int32),
        pltpu.VMEM((_CPW, _CHUNK), jnp.int32),
        pltpu.VMEM((_NBUF, _CHUNK, _NC), jnp.float32),
        pltpu.VMEM((_RPW, _NC), jnp.float32),
        pltpu.VMEM((_RPW, _NC), jnp.float32),
        pltpu.VMEM((_RPW, _NC), jnp.float32),
        pltpu.VMEM((_RPW, _NC), jnp.float32),
        pltpu.VMEM((_RPW, _NC), jnp.float32),
        pltpu.VMEM((_ZPT, _NC), jnp.float32),
        pltpu.VMEM((16, 16), jnp.float32),
        pltpu.VMEM_SHARED((_NPAD, _NC), jnp.float32),
        pltpu.SemaphoreType.DMA((_NBUF,)),
        pltpu.SemaphoreType.DMA((_NBUF,)),
        pltpu.SemaphoreType.REGULAR,
    ],
    compiler_params=pltpu.CompilerParams(use_tc_tiling_on_sc=False),
)


def kernel(x, edge_index, W1, b1, W2, b2, temp):
    x = x.astype(jnp.float32)
    row = edge_index[0].astype(jnp.int32)
    col = edge_index[1].astype(jnp.int32)

    nfk = _EPW - _EREAL
    fk = (jnp.arange(_NW * nfk, dtype=jnp.int32) % (_NPAD - _N)) + _N
    fk = fk.reshape(_NW, nfk)
    rows = jnp.concatenate([row.reshape(_NW, _EREAL), fk], axis=1)
    cols = jnp.concatenate([col.reshape(_NW, _EREAL), fk], axis=1)
    rows = rows.reshape(_NW, _CPW, _CHUNK)
    cols = cols.reshape(_NW, _CPW, _CHUNK)

    xp = jnp.pad(x, ((0, _NPAD - _N), (0, 0)))
    temp16 = jnp.pad(temp.astype(jnp.float32), (0, 16 - (_K + 1)))
    tempb = jnp.broadcast_to(temp16[:, None], (16, 16))

    h0 = _mlp(xp, W1, b1.reshape(1, _NH), W2, b2.reshape(1, _NC))
    hidden, _, _ = _scprop(rows, cols, h0, tempb)
    return _logsoftmax(hidden[:_N])


# speedup vs baseline: 1.0467x; 1.0467x over previous
import jax
import jax.numpy as jnp
from jax import lax
from jax.experimental import pallas as pl
from jax.experimental.pallas import tpu as pltpu
from jax.experimental.pallas import tpu_sc as plsc

_N = 10000
_E = 320000
_NF = 128
_NH = 256
_NC = 16
_K = 10

_NS = 16
_NW = 32
_NPAD = 10240
_RPW = _NPAD // _NW
_ZPT = _NPAD // _NS
_CHUNK = 128
_CPW = 80
_EPW = _CPW * _CHUNK
_EREAL = _E // _NW
_NBUF = 8
_GRP = _CPW // _NBUF


def _mlp_body(x_ref, w1_ref, b1_ref, w2_ref, b2_ref, o_ref):
    h = jnp.dot(x_ref[...], w1_ref[...], preferred_element_type=jnp.float32)
    h = jnp.maximum(h + b1_ref[...], 0.0)
    o_ref[...] = jnp.dot(h, w2_ref[...], preferred_element_type=jnp.float32) + b2_ref[...]


_MLP_BLK = 1024


def _mlp(xp, W1, b1, W2, b2):
    return pl.pallas_call(
        _mlp_body,
        grid=(_NPAD // _MLP_BLK,),
        in_specs=[
            pl.BlockSpec((_MLP_BLK, _NF), lambda i: (i, 0)),
            pl.BlockSpec((_NF, _NH), lambda i: (0, 0)),
            pl.BlockSpec((1, _NH), lambda i: (0, 0)),
            pl.BlockSpec((_NH, _NC), lambda i: (0, 0)),
            pl.BlockSpec((1, _NC), lambda i: (0, 0)),
        ],
        out_specs=pl.BlockSpec((_MLP_BLK, _NC), lambda i: (i, 0)),
        out_shape=jax.ShapeDtypeStruct((_NPAD, _NC), jnp.float32),
    )(xp, W1, b1, W2, b2)


def _lsm_body(h_ref, o_ref):
    v = h_ref[...]
    m = jnp.max(v, axis=1, keepdims=True)
    e = jnp.exp(v - m)
    s = jnp.sum(e, axis=1, keepdims=True)
    o_ref[...] = (v - m) - jnp.log(s)


_LSM_BLK = 2000


def _logsoftmax(h):
    return pl.pallas_call(
        _lsm_body,
        grid=(_N // _LSM_BLK,),
        in_specs=[pl.BlockSpec((_LSM_BLK, _NC), lambda i: (i, 0))],
        out_specs=pl.BlockSpec((_LSM_BLK, _NC), lambda i: (i, 0)),
        out_shape=jax.ShapeDtypeStruct((_N, _NC), jnp.float32),
    )(h)


def _sc_body(rows_hbm, cols_hbm, h0_hbm, temp_hbm,
             hidden_hbm, gwork_hbm, part_hbm,
             row_t, col_t, bufs, g_t, acc_t, d2_t, sbuf_t, pbuf_t, zero_t,
             temp_v, s_sh, gsem, ssem, xsem):
    cid = lax.axis_index("c")
    sid = lax.axis_index("s")
    w = cid * _NS + sid
    r0 = w * _RPW
    own = pl.ds(r0, _RPW)
    pr0 = ((1 - cid) * _NS + sid) * _RPW
    peer = pl.ds(pr0, _RPW)
    zrows = pl.ds(sid * _ZPT, _ZPT)

    def xbarrier():
        plsc.subcore_barrier()

        @pl.when(sid == 0)
        def _x():
            pltpu.semaphore_signal(xsem, 1, core_index=1 - cid)
            pl.semaphore_wait(xsem, 1)
        plsc.subcore_barrier()

    pltpu.sync_copy(rows_hbm.at[w], row_t)
    pltpu.sync_copy(cols_hbm.at[w], col_t)
    pltpu.sync_copy(temp_hbm, temp_v)

    ones = jnp.ones((_NC,), jnp.float32)
    zeros = jnp.zeros((_NC,), jnp.float32)

    def _fill_zero(i, c):
        zero_t[i, :] = zeros
        return c

    def _fill_one(i, c):
        zero_t[i, :] = ones
        return c

    def _fill_src_ones(i, c):
        sbuf_t[i, :] = ones
        return c

    lax.fori_loop(0, _CHUNK, _fill_src_ones, 0)

    @pl.when(cid == 0)
    def _i0():
        lax.fori_loop(0, _ZPT, _fill_one, 0)

    @pl.when(cid == 1)
    def _i1():
        lax.fori_loop(0, _ZPT, _fill_zero, 0)
    pltpu.sync_copy(zero_t, s_sh.at[zrows])

    @pl.when(cid == 0)
    def _i0z():
        lax.fori_loop(0, _ZPT, _fill_zero, 0)
    plsc.subcore_barrier()

    def _deg_group(m, c):
        for b in range(_NBUF):
            j = m * _NBUF + b
            pltpu.async_copy(sbuf_t.at[pl.ds(0, _CHUNK)],
                             s_sh.at[col_t.at[j]], ssem.at[0], add=True)
        for b in range(_NBUF):
            j = m * _NBUF + b
            pltpu.make_async_copy(sbuf_t.at[pl.ds(0, _CHUNK)],
                                  s_sh.at[col_t.at[j]], ssem.at[0]).wait()
        return c
    lax.fori_loop(0, _GRP, _deg_group, 0)
    plsc.subcore_barrier()

    pltpu.async_copy(s_sh.at[peer], part_hbm.at[peer], gsem.at[0])
    pltpu.async_copy(s_sh.at[own], sbuf_t, gsem.at[1])
    pltpu.make_async_copy(s_sh.at[peer], part_hbm.at[peer], gsem.at[0]).wait()
    pltpu.make_async_copy(s_sh.at[own], sbuf_t, gsem.at[1]).wait()
    pltpu.async_copy(zero_t.at[pl.ds(0, _RPW)], s_sh.at[own], gsem.at[2])
    pltpu.async_copy(zero_t.at[pl.ds(0, _RPW)], s_sh.at[peer], gsem.at[3])
    pltpu.make_async_copy(zero_t.at[pl.ds(0, _RPW)], s_sh.at[own], gsem.at[2]).wait()
    pltpu.make_async_copy(zero_t.at[pl.ds(0, _RPW)], s_sh.at[peer], gsem.at[3]).wait()
    xbarrier()

    pltpu.sync_copy(part_hbm.at[own], pbuf_t)
    pltpu.sync_copy(h0_hbm.at[own], g_t)
    t0 = temp_v[0, :]

    def _prep(i, c):
        d = sbuf_t[i, :] + pbuf_t[i, :]
        y = 1.0 / d
        for _ in range(20):
            y = y * (1.5 - 0.5 * d * y * y)
        d2_t[i, :] = y * y
        g0 = y * g_t[i, :]
        g_t[i, :] = g0
        acc_t[i, :] = t0 * g0
        return c
    lax.fori_loop(0, _RPW, _prep, 0)
    pltpu.sync_copy(g_t, gwork_hbm.at[own])
    xbarrier()

    def _hop(k, c):
        for b in range(_NBUF):
            pltpu.async_copy(gwork_hbm.at[row_t.at[b]], bufs.at[b], gsem.at[b])

        def _group(m, cc):
            for b in range(_NBUF):
                j = m * _NBUF + b
                pltpu.make_async_copy(gwork_hbm.at[row_t.at[j]],
                                      bufs.at[b], gsem.at[b]).wait()
                pltpu.async_copy(bufs.at[b], s_sh.at[col_t.at[j]],
                                 ssem.at[b], add=True)
            for b in range(_NBUF):
                j = m * _NBUF + b

                @pl.when(m < _GRP - 1)
                def _refill():
                    pltpu.make_async_copy(bufs.at[b], s_sh.at[col_t.at[j]],
                                          ssem.at[b]).wait()
                    pltpu.async_copy(gwork_hbm.at[row_t.at[j + _NBUF]],
                                     bufs.at[b], gsem.at[b])
            return cc
        lax.fori_loop(0, _GRP, _group, 0)
        for b in range(_NBUF):
            j = (_GRP - 1) * _NBUF + b
            pltpu.make_async_copy(bufs.at[b], s_sh.at[col_t.at[j]],
                                  ssem.at[b]).wait()
        plsc.subcore_barrier()

        pltpu.async_copy(s_sh.at[peer], part_hbm.at[peer], gsem.at[0])
        pltpu.async_copy(s_sh.at[own], sbuf_t, gsem.at[1])
        pltpu.make_async_copy(s_sh.at[peer], part_hbm.at[peer],
                              gsem.at[0]).wait()
        pltpu.make_async_copy(s_sh.at[own], sbuf_t, gsem.at[1]).wait()
        pltpu.async_copy(zero_t.at[pl.ds(0, _RPW)], s_sh.at[own], gsem.at[2])
        pltpu.async_copy(zero_t.at[pl.ds(0, _RPW)], s_sh.at[peer], gsem.at[3])
        pltpu.make_async_copy(zero_t.at[pl.ds(0, _RPW)], s_sh.at[own], gsem.at[2]).wait()
        pltpu.make_async_copy(zero_t.at[pl.ds(0, _RPW)], s_sh.at[peer], gsem.at[3]).wait()
        xbarrier()

        pltpu.sync_copy(part_hbm.at[own], pbuf_t)
        tk = temp_v[k + 1, :]

        def _upd(i, cc):
            gn = d2_t[i, :] * (sbuf_t[i, :] + pbuf_t[i, :] + g_t[i, :])
            acc_t[i, :] = acc_t[i, :] + tk * gn
            g_t[i, :] = gn
            return cc
        lax.fori_loop(0, _RPW, _upd, 0)
        pltpu.sync_copy(g_t, gwork_hbm.at[own])
        xbarrier()
        return c
    lax.fori_loop(0, _K, _hop, 0)

    def _fin(i, c):
        d2 = d2_t[i, :]
        y = jnp.ones((_NC,), jnp.float32)
        for _ in range(20):
            y = y * (1.5 - 0.5 * d2 * y * y)
        acc_t[i, :] = y * acc_t[i, :]
        return c
    lax.fori_loop(0, _RPW, _fin, 0)
    pltpu.sync_copy(acc_t, hidden_hbm.at[own])


_sc_mesh = plsc.VectorSubcoreMesh(core_axis_name="c", subcore_axis_name="s",
                                  num_cores=2, num_subcores=_NS)

_scprop = pl.kernel(
    _sc_body,
    out_type=(jax.ShapeDtypeStruct((_NPAD, _NC), jnp.float32),
              jax.ShapeDtypeStruct((_NPAD, _NC), jnp.float32),
              jax.ShapeDtypeStruct((_NPAD, _NC), jnp.float32)),
    mesh=_sc_mesh,
    scratch_types=[
        pltpu.VMEM((_CPW, _CHUNK), jnp.int32),
        pltpu.VMEM((_CPW, _CHUNK), jnp.int32),
        pltpu.VMEM((_NBUF, _CHUNK, _NC), jnp.float32),
        pltpu.VMEM((_RPW, _NC), jnp.float32),
        pltpu.VMEM((_RPW, _NC), jnp.float32),
        pltpu.VMEM((_RPW, _NC), jnp.float32),
        pltpu.VMEM((_RPW, _NC), jnp.float32),
        pltpu.VMEM((_RPW, _NC), jnp.float32),
        pltpu.VMEM((_ZPT, _NC), jnp.float32),
        pltpu.VMEM((16, 16), jnp.float32),
        pltpu.VMEM_SHARED((_NPAD, _NC), jnp.float32),
        pltpu.SemaphoreType.DMA((_NBUF,)),
        pltpu.SemaphoreType.DMA((_NBUF,)),
        pltpu.SemaphoreType.REGULAR,
    ],
    compiler_params=pltpu.CompilerParams(use_tc_tiling_on_sc=False),
)


def kernel(x, edge_index, W1, b1, W2, b2, temp):
    x = x.astype(jnp.float32)
    row = edge_index[0].astype(jnp.int32)
    col = edge_index[1].astype(jnp.int32)

    nfk = _EPW - _EREAL
    fk = (jnp.arange(_NW * nfk, dtype=jnp.int32) % (_NPAD - _N)) + _N
    fk = fk.reshape(_NW, nfk)
    rows = jnp.concatenate([row.reshape(_NW, _EREAL), fk], axis=1)
    cols = jnp.concatenate([col.reshape(_NW, _EREAL), fk], axis=1)
    rows = rows.reshape(_NW, _CPW, _CHUNK)
    cols = cols.reshape(_NW, _CPW, _CHUNK)

    xp = jnp.pad(x, ((0, _NPAD - _N), (0, 0)))
    temp16 = jnp.pad(temp.astype(jnp.float32), (0, 16 - (_K + 1)))
    tempb = jnp.broadcast_to(temp16[:, None], (16, 16))

    h0 = _mlp(xp, W1, b1.reshape(1, _NH), W2, b2.reshape(1, _NC))
    hidden, _, _ = _scprop(rows, cols, h0, tempb)
    return _logsoftmax(hidden[:_N])


# speedup vs baseline: 1.0612x; 1.0138x over previous
import jax
import jax.numpy as jnp
from jax import lax
from jax.experimental import pallas as pl
from jax.experimental.pallas import tpu as pltpu
from jax.experimental.pallas import tpu_sc as plsc

_N = 10000
_E = 320000
_NF = 128
_NH = 256
_NC = 16
_K = 10

_NS = 16
_NW = 32
_NPAD = 10240
_RPW = _NPAD // _NW
_ZPT = _NPAD // _NS
_CHUNK = 128
_CPW = 80
_EPW = _CPW * _CHUNK
_EREAL = _E // _NW
_NBUF = 10
_GRP = _CPW // _NBUF


def _mlp_body(x_ref, w1_ref, b1_ref, w2_ref, b2_ref, o_ref):
    h = jnp.dot(x_ref[...], w1_ref[...], preferred_element_type=jnp.float32)
    h = jnp.maximum(h + b1_ref[...], 0.0)
    o_ref[...] = jnp.dot(h, w2_ref[...], preferred_element_type=jnp.float32) + b2_ref[...]


_MLP_BLK = 1024


def _mlp(xp, W1, b1, W2, b2):
    return pl.pallas_call(
        _mlp_body,
        grid=(_NPAD // _MLP_BLK,),
        in_specs=[
            pl.BlockSpec((_MLP_BLK, _NF), lambda i: (i, 0)),
            pl.BlockSpec((_NF, _NH), lambda i: (0, 0)),
            pl.BlockSpec((1, _NH), lambda i: (0, 0)),
            pl.BlockSpec((_NH, _NC), lambda i: (0, 0)),
            pl.BlockSpec((1, _NC), lambda i: (0, 0)),
        ],
        out_specs=pl.BlockSpec((_MLP_BLK, _NC), lambda i: (i, 0)),
        out_shape=jax.ShapeDtypeStruct((_NPAD, _NC), jnp.float32),
    )(xp, W1, b1, W2, b2)


def _lsm_body(h_ref, o_ref):
    v = h_ref[...]
    m = jnp.max(v, axis=1, keepdims=True)
    e = jnp.exp(v - m)
    s = jnp.sum(e, axis=1, keepdims=True)
    o_ref[...] = (v - m) - jnp.log(s)


_LSM_BLK = 2000


def _logsoftmax(h):
    return pl.pallas_call(
        _lsm_body,
        grid=(_N // _LSM_BLK,),
        in_specs=[pl.BlockSpec((_LSM_BLK, _NC), lambda i: (i, 0))],
        out_specs=pl.BlockSpec((_LSM_BLK, _NC), lambda i: (i, 0)),
        out_shape=jax.ShapeDtypeStruct((_N, _NC), jnp.float32),
    )(h)


def _sc_body(rows_hbm, cols_hbm, h0_hbm, temp_hbm,
             hidden_hbm, gwork_hbm, part_hbm,
             row_t, col_t, bufs, g_t, acc_t, d2_t, sbuf_t, pbuf_t, zero_t,
             temp_v, s_sh, gsem, ssem, xsem):
    cid = lax.axis_index("c")
    sid = lax.axis_index("s")
    w = cid * _NS + sid
    r0 = w * _RPW
    own = pl.ds(r0, _RPW)
    pr0 = ((1 - cid) * _NS + sid) * _RPW
    peer = pl.ds(pr0, _RPW)
    zrows = pl.ds(sid * _ZPT, _ZPT)

    def xbarrier():
        plsc.subcore_barrier()

        @pl.when(sid == 0)
        def _x():
            pltpu.semaphore_signal(xsem, 1, core_index=1 - cid)
            pl.semaphore_wait(xsem, 1)
        plsc.subcore_barrier()

    pltpu.sync_copy(rows_hbm.at[w], row_t)
    pltpu.sync_copy(cols_hbm.at[w], col_t)
    pltpu.sync_copy(temp_hbm, temp_v)

    ones = jnp.ones((_NC,), jnp.float32)
    zeros = jnp.zeros((_NC,), jnp.float32)

    def _fill_zero(i, c):
        zero_t[i, :] = zeros
        return c

    def _fill_one(i, c):
        zero_t[i, :] = ones
        return c

    def _fill_src_ones(i, c):
        sbuf_t[i, :] = ones
        return c

    lax.fori_loop(0, _CHUNK, _fill_src_ones, 0)

    @pl.when(cid == 0)
    def _i0():
        lax.fori_loop(0, _ZPT, _fill_one, 0)

    @pl.when(cid == 1)
    def _i1():
        lax.fori_loop(0, _ZPT, _fill_zero, 0)
    pltpu.sync_copy(zero_t, s_sh.at[zrows])

    @pl.when(cid == 0)
    def _i0z():
        lax.fori_loop(0, _ZPT, _fill_zero, 0)
    plsc.subcore_barrier()

    def _deg_group(m, c):
        for b in range(_NBUF):
            j = m * _NBUF + b
            pltpu.async_copy(sbuf_t.at[pl.ds(0, _CHUNK)],
                             s_sh.at[col_t.at[j]], ssem.at[0], add=True)
        for b in range(_NBUF):
            j = m * _NBUF + b
            pltpu.make_async_copy(sbuf_t.at[pl.ds(0, _CHUNK)],
                                  s_sh.at[col_t.at[j]], ssem.at[0]).wait()
        return c
    lax.fori_loop(0, _GRP, _deg_group, 0)
    plsc.subcore_barrier()

    pltpu.async_copy(s_sh.at[peer], part_hbm.at[peer], gsem.at[0])
    pltpu.async_copy(s_sh.at[own], sbuf_t, gsem.at[1])
    pltpu.make_async_copy(s_sh.at[peer], part_hbm.at[peer], gsem.at[0]).wait()
    pltpu.make_async_copy(s_sh.at[own], sbuf_t, gsem.at[1]).wait()
    pltpu.async_copy(zero_t.at[pl.ds(0, _RPW)], s_sh.at[own], gsem.at[2])
    pltpu.async_copy(zero_t.at[pl.ds(0, _RPW)], s_sh.at[peer], gsem.at[3])
    pltpu.make_async_copy(zero_t.at[pl.ds(0, _RPW)], s_sh.at[own], gsem.at[2]).wait()
    pltpu.make_async_copy(zero_t.at[pl.ds(0, _RPW)], s_sh.at[peer], gsem.at[3]).wait()
    xbarrier()

    pltpu.sync_copy(part_hbm.at[own], pbuf_t)
    pltpu.sync_copy(h0_hbm.at[own], g_t)
    t0 = temp_v[0, :]

    def _prep(i, c):
        d = sbuf_t[i, :] + pbuf_t[i, :]
        y = 1.0 / d
        for _ in range(20):
            y = y * (1.5 - 0.5 * d * y * y)
        d2_t[i, :] = y * y
        g0 = y * g_t[i, :]
        g_t[i, :] = g0
        acc_t[i, :] = t0 * g0
        return c
    lax.fori_loop(0, _RPW, _prep, 0)
    pltpu.sync_copy(g_t, gwork_hbm.at[own])
    xbarrier()

    def _hop(k, c):
        for b in range(_NBUF):
            pltpu.async_copy(gwork_hbm.at[row_t.at[b]], bufs.at[b], gsem.at[b])

        def _group(m, cc):
            for b in range(_NBUF):
                j = m * _NBUF + b
                pltpu.make_async_copy(gwork_hbm.at[row_t.at[j]],
                                      bufs.at[b], gsem.at[b]).wait()
                pltpu.async_copy(bufs.at[b], s_sh.at[col_t.at[j]],
                                 ssem.at[b], add=True)
            for b in range(_NBUF):
                j = m * _NBUF + b

                @pl.when(m < _GRP - 1)
                def _refill():
                    pltpu.make_async_copy(bufs.at[b], s_sh.at[col_t.at[j]],
                                          ssem.at[b]).wait()
                    pltpu.async_copy(gwork_hbm.at[row_t.at[j + _NBUF]],
                                     bufs.at[b], gsem.at[b])
            return cc
        lax.fori_loop(0, _GRP, _group, 0)
        for b in range(_NBUF):
            j = (_GRP - 1) * _NBUF + b
            pltpu.make_async_copy(bufs.at[b], s_sh.at[col_t.at[j]],
                                  ssem.at[b]).wait()
        plsc.subcore_barrier()

        pltpu.async_copy(s_sh.at[peer], part_hbm.at[peer], gsem.at[0])
        pltpu.async_copy(s_sh.at[own], sbuf_t, gsem.at[1])
        pltpu.make_async_copy(s_sh.at[peer], part_hbm.at[peer],
                              gsem.at[0]).wait()
        pltpu.make_async_copy(s_sh.at[own], sbuf_t, gsem.at[1]).wait()
        pltpu.async_copy(zero_t.at[pl.ds(0, _RPW)], s_sh.at[own], gsem.at[2])
        pltpu.async_copy(zero_t.at[pl.ds(0, _RPW)], s_sh.at[peer], gsem.at[3])
        pltpu.make_async_copy(zero_t.at[pl.ds(0, _RPW)], s_sh.at[own], gsem.at[2]).wait()
        pltpu.make_async_copy(zero_t.at[pl.ds(0, _RPW)], s_sh.at[peer], gsem.at[3]).wait()
        xbarrier()

        pltpu.sync_copy(part_hbm.at[own], pbuf_t)
        tk = temp_v[k + 1, :]

        def _upd(i, cc):
            gn = d2_t[i, :] * (sbuf_t[i, :] + pbuf_t[i, :] + g_t[i, :])
            acc_t[i, :] = acc_t[i, :] + tk * gn
            g_t[i, :] = gn
            return cc
        lax.fori_loop(0, _RPW, _upd, 0)
        pltpu.sync_copy(g_t, gwork_hbm.at[own])
        xbarrier()
        return c
    lax.fori_loop(0, _K, _hop, 0)

    def _fin(i, c):
        d2 = d2_t[i, :]
        y = jnp.ones((_NC,), jnp.float32)
        for _ in range(20):
            y = y * (1.5 - 0.5 * d2 * y * y)
        acc_t[i, :] = y * acc_t[i, :]
        return c
    lax.fori_loop(0, _RPW, _fin, 0)
    pltpu.sync_copy(acc_t, hidden_hbm.at[own])


_sc_mesh = plsc.VectorSubcoreMesh(core_axis_name="c", subcore_axis_name="s",
                                  num_cores=2, num_subcores=_NS)

_scprop = pl.kernel(
    _sc_body,
    out_type=(jax.ShapeDtypeStruct((_NPAD, _NC), jnp.float32),
              jax.ShapeDtypeStruct((_NPAD, _NC), jnp.float32),
              jax.ShapeDtypeStruct((_NPAD, _NC), jnp.float32)),
    mesh=_sc_mesh,
    scratch_types=[
        pltpu.VMEM((_CPW, _CHUNK), jnp.int32),
        pltpu.VMEM((_CPW, _CHUNK), jnp.int32),
        pltpu.VMEM((_NBUF, _CHUNK, _NC), jnp.float32),
        pltpu.VMEM((_RPW, _NC), jnp.float32),
        pltpu.VMEM((_RPW, _NC), jnp.float32),
        pltpu.VMEM((_RPW, _NC), jnp.float32),
        pltpu.VMEM((_RPW, _NC), jnp.float32),
        pltpu.VMEM((_RPW, _NC), jnp.float32),
        pltpu.VMEM((_ZPT, _NC), jnp.float32),
        pltpu.VMEM((16, 16), jnp.float32),
        pltpu.VMEM_SHARED((_NPAD, _NC), jnp.float32),
        pltpu.SemaphoreType.DMA((_NBUF,)),
        pltpu.SemaphoreType.DMA((_NBUF,)),
        pltpu.SemaphoreType.REGULAR,
    ],
    compiler_params=pltpu.CompilerParams(use_tc_tiling_on_sc=False),
)


def kernel(x, edge_index, W1, b1, W2, b2, temp):
    x = x.astype(jnp.float32)
    row = edge_index[0].astype(jnp.int32)
    col = edge_index[1].astype(jnp.int32)

    nfk = _EPW - _EREAL
    fk = (jnp.arange(_NW * nfk, dtype=jnp.int32) % (_NPAD - _N)) + _N
    fk = fk.reshape(_NW, nfk)
    rows = jnp.concatenate([row.reshape(_NW, _EREAL), fk], axis=1)
    cols = jnp.concatenate([col.reshape(_NW, _EREAL), fk], axis=1)
    rows = rows.reshape(_NW, _CPW, _CHUNK)
    cols = cols.reshape(_NW, _CPW, _CHUNK)

    xp = jnp.pad(x, ((0, _NPAD - _N), (0, 0)))
    temp16 = jnp.pad(temp.astype(jnp.float32), (0, 16 - (_K + 1)))
    tempb = jnp.broadcast_to(temp16[:, None], (16, 16))

    h0 = _mlp(xp, W1, b1.reshape(1, _NH), W2, b2.reshape(1, _NC))
    hidden, _, _ = _scprop(rows, cols, h0, tempb)
    return _logsoftmax(hidden[:_N])
